# baseline (speedup 1.0000x reference)
import functools

import jax
import jax.numpy as jnp
from jax.experimental import pallas as pl
from jax.experimental.pallas import tpu as pltpu

NUM_TOKENS = 8192
EMBED_DIM = 2048
NUM_EXPERTS = 16
TOP_K = 2
BLOCK_N = 1024


def _top2(probs):
    cols = jax.lax.broadcasted_iota(jnp.int32, probs.shape, 1)
    i1 = jnp.argmax(probs, axis=-1).astype(jnp.int32)
    p1 = jnp.max(probs, axis=-1)
    masked = jnp.where(cols == i1[:, None], -jnp.inf, probs)
    i2 = jnp.argmax(masked, axis=-1).astype(jnp.int32)
    p2 = jnp.max(masked, axis=-1)
    idx = jnp.concatenate([i1[:, None], i2[:, None]], axis=1)
    wgt = jnp.concatenate([p1[:, None], p2[:, None]], axis=1)
    return idx, wgt


def _gate_body(x_ref, wt_ref, idx_ref, wgt_ref, row_ref):
    i = pl.program_id(0)
    logits = jnp.dot(x_ref[...], wt_ref[...],
                     preferred_element_type=jnp.float32)
    m = jnp.max(logits, axis=-1, keepdims=True)
    e = jnp.exp(logits - m)
    probs = e / jnp.sum(e, axis=-1, keepdims=True)
    idx, wgt = _top2(probs)

    base = i * BLOCK_N
    t = base + jax.lax.broadcasted_iota(jnp.int32, (BLOCK_N, 1), 0)
    idx_ref[pl.ds(base, BLOCK_N), :] = idx
    wgt_ref[pl.ds(base, BLOCK_N), :] = wgt
    row_ref[pl.ds(base, BLOCK_N), :] = jnp.concatenate([t, t + NUM_TOKENS],
                                                       axis=1)


@functools.partial(jax.jit, static_argnames=())
def kernel(hidden_states, weight):
    n, d = hidden_states.shape
    wt = weight.T
    out = pl.pallas_call(
        _gate_body,
        grid=(n // BLOCK_N,),
        in_specs=[
            pl.BlockSpec((BLOCK_N, d), lambda i: (i, 0)),
            pl.BlockSpec((d, NUM_EXPERTS), lambda i: (0, 0)),
        ],
        out_specs=[
            pl.BlockSpec((n, TOP_K), lambda i: (0, 0)),
            pl.BlockSpec((n, TOP_K), lambda i: (0, 0)),
            pl.BlockSpec((n, TOP_K), lambda i: (0, 0)),
        ],
        out_shape=[
            jax.ShapeDtypeStruct((n, TOP_K), jnp.int32),
            jax.ShapeDtypeStruct((n, TOP_K), jnp.float32),
            jax.ShapeDtypeStruct((n, TOP_K), jnp.int32),
        ],
        compiler_params=pltpu.CompilerParams(
            dimension_semantics=("arbitrary",),
        ),
    )(hidden_states, wt)
    return out[0], out[1], out[2]


# speedup vs baseline: 1.0049x; 1.0049x over previous
import functools

import jax
import jax.numpy as jnp
from jax.experimental import pallas as pl
from jax.experimental.pallas import tpu as pltpu

NUM_TOKENS = 8192
EMBED_DIM = 2048
NUM_EXPERTS = 16
TOP_K = 2
BLOCK_N = 512
NBUF = 4
NSTEPS = NUM_TOKENS // BLOCK_N


def _top2(probs):
    cols = jax.lax.broadcasted_iota(jnp.int32, probs.shape, 1)
    i1 = jnp.argmax(probs, axis=-1).astype(jnp.int32)
    p1 = jnp.max(probs, axis=-1)
    masked = jnp.where(cols == i1[:, None], -jnp.inf, probs)
    i2 = jnp.argmax(masked, axis=-1).astype(jnp.int32)
    p2 = jnp.max(masked, axis=-1)
    idx = jnp.concatenate([i1[:, None], i2[:, None]], axis=1)
    wgt = jnp.concatenate([p1[:, None], p2[:, None]], axis=1)
    return idx, wgt


def _gate_body(x_hbm, wt_ref, idx_ref, wgt_ref, row_ref, *scratch):
    bufs = scratch[:NBUF]
    sems = scratch[NBUF:]
    i = pl.program_id(0)

    def copy(step, buf, sem):
        return pltpu.make_async_copy(
            x_hbm.at[pl.ds(step * BLOCK_N, BLOCK_N), :], buf, sem)

    @pl.when(i == 0)
    def _warmup():
        for s in range(NBUF):
            copy(s, bufs[s], sems[s]).start()

    def run_slot(slot):
        copy(i, bufs[slot], sems[slot]).wait()
        logits = jnp.dot(bufs[slot][...], wt_ref[...],
                         preferred_element_type=jnp.float32)
        m = jnp.max(logits, axis=-1, keepdims=True)
        e = jnp.exp(logits - m)
        probs = e / jnp.sum(e, axis=-1, keepdims=True)
        idx, wgt = _top2(probs)
        base = i * BLOCK_N
        t = base + jax.lax.broadcasted_iota(jnp.int32, (BLOCK_N, 1), 0)
        idx_ref[pl.ds(base, BLOCK_N), :] = idx
        wgt_ref[pl.ds(base, BLOCK_N), :] = wgt
        row_ref[pl.ds(base, BLOCK_N), :] = jnp.concatenate(
            [t, t + NUM_TOKENS], axis=1)

        @pl.when(i + NBUF < NSTEPS)
        def _refill():
            copy(i + NBUF, bufs[slot], sems[slot]).start()

    for s in range(NBUF):
        @pl.when(i % NBUF == s)
        def _dispatch(s=s):
            run_slot(s)


@functools.partial(jax.jit, static_argnames=())
def kernel(hidden_states, weight):
    n, d = hidden_states.shape
    wt = weight.T
    out = pl.pallas_call(
        _gate_body,
        grid=(NSTEPS,),
        in_specs=[
            pl.BlockSpec(memory_space=pl.ANY),
            pl.BlockSpec((d, NUM_EXPERTS), lambda i: (0, 0)),
        ],
        out_specs=[
            pl.BlockSpec((n, TOP_K), lambda i: (0, 0)),
            pl.BlockSpec((n, TOP_K), lambda i: (0, 0)),
            pl.BlockSpec((n, TOP_K), lambda i: (0, 0)),
        ],
        out_shape=[
            jax.ShapeDtypeStruct((n, TOP_K), jnp.int32),
            jax.ShapeDtypeStruct((n, TOP_K), jnp.float32),
            jax.ShapeDtypeStruct((n, TOP_K), jnp.int32),
        ],
        scratch_shapes=(
            [pltpu.VMEM((BLOCK_N, EMBED_DIM), jnp.float32)] * NBUF
            + [pltpu.SemaphoreType.DMA] * NBUF
        ),
        compiler_params=pltpu.CompilerParams(
            dimension_semantics=("arbitrary",),
        ),
    )(hidden_states, wt)
    return out[0], out[1], out[2]
